# baseline (speedup 1.0000x reference)
import jax
import jax.numpy as jnp
from jax.experimental import pallas as pl
from jax.experimental.pallas import tpu as pltpu

TOK = 512
NSTREAM = 4
CHUNK = TOK * NSTREAM


def _router_kernel(x_hbm, w_ref, o_ref, xb, sems):
    i = pl.program_id(0)
    nstep = pl.num_programs(0)

    def issue(step, slot):
        base = step * CHUNK
        for s in range(NSTREAM):
            pltpu.make_async_copy(
                x_hbm.at[pl.ds(base + s * TOK, TOK), :],
                xb.at[slot, s],
                sems.at[slot, s],
            ).start()

    @pl.when(i == 0)
    def _():
        issue(0, 0)

    @pl.when(i + 1 < nstep)
    def _():
        issue(i + 1, jax.lax.rem(i + 1, 2))

    slot = jax.lax.rem(i, 2)
    for s in range(NSTREAM):
        pltpu.make_async_copy(
            x_hbm.at[pl.ds(i * CHUNK + s * TOK, TOK), :],
            xb.at[slot, s],
            sems.at[slot, s],
        ).wait()
    w = w_ref[...]
    dims = (((1,), (1,)), ((), ()))
    for s in range(NSTREAM):
        o_ref[pl.ds(s * TOK, TOK), :] = jax.lax.dot_general(
            xb[slot, s], w, dimension_numbers=dims,
            preferred_element_type=jnp.float32)


def kernel(hidden_states, W):
    B, S, H = hidden_states.shape
    E = W.shape[0]
    T = B * S
    x = hidden_states.reshape(T, H)
    out = pl.pallas_call(
        _router_kernel,
        grid=(T // CHUNK,),
        in_specs=[
            pl.BlockSpec(memory_space=pltpu.MemorySpace.HBM),
            pl.BlockSpec((E, H), lambda i: (0, 0)),
        ],
        out_specs=pl.BlockSpec((CHUNK, E), lambda i: (i, 0)),
        out_shape=jax.ShapeDtypeStruct((T, E), jnp.float32),
        scratch_shapes=[
            pltpu.VMEM((2, NSTREAM, TOK, H), jnp.float32),
            pltpu.SemaphoreType.DMA((2, NSTREAM)),
        ],
        compiler_params=pltpu.CompilerParams(
            dimension_semantics=("arbitrary",),
        ),
    )(x, W)
    return out.reshape(B, S, E)


# speedup vs baseline: 1.0163x; 1.0163x over previous
import jax
import jax.numpy as jnp
from jax.experimental import pallas as pl
from jax.experimental.pallas import tpu as pltpu

TOK_BLK = 1024
NSEG = 2


def _router_kernel(x_ref, w_ref, o_ref):
    w = w_ref[...]
    dims = (((1,), (1,)), ((), ()))
    for s in range(NSEG):
        o_ref[s] = jax.lax.dot_general(
            x_ref[s], w, dimension_numbers=dims,
            preferred_element_type=jnp.float32)


def kernel(hidden_states, W):
    B, S, H = hidden_states.shape
    E = W.shape[0]
    T = B * S
    x = hidden_states.reshape(NSEG, T // NSEG, H)
    out = pl.pallas_call(
        _router_kernel,
        grid=(T // NSEG // TOK_BLK,),
        in_specs=[
            pl.BlockSpec((NSEG, TOK_BLK, H), lambda i: (0, i, 0)),
            pl.BlockSpec((E, H), lambda i: (0, 0)),
        ],
        out_specs=pl.BlockSpec((NSEG, TOK_BLK, E), lambda i: (0, i, 0)),
        out_shape=jax.ShapeDtypeStruct((NSEG, T // NSEG, E), jnp.float32),
        compiler_params=pltpu.CompilerParams(
            dimension_semantics=("arbitrary",),
        ),
    )(x, W)
    return out.reshape(B, S, E)


# speedup vs baseline: 1.0169x; 1.0006x over previous
import jax
import jax.numpy as jnp
from jax.experimental import pallas as pl
from jax.experimental.pallas import tpu as pltpu

TOK_BLK = 2048


def _router_kernel(x_ref, w_ref, o_ref):
    o_ref[0] = jax.lax.dot_general(
        x_ref[0], w_ref[...],
        dimension_numbers=(((1,), (1,)), ((), ())),
        preferred_element_type=jnp.float32)


def kernel(hidden_states, W):
    B, S, H = hidden_states.shape
    E = W.shape[0]
    return pl.pallas_call(
        _router_kernel,
        grid=(B, S // TOK_BLK),
        in_specs=[
            pl.BlockSpec((1, TOK_BLK, H), lambda b, i: (b, i, 0)),
            pl.BlockSpec((E, H), lambda b, i: (0, 0)),
        ],
        out_specs=pl.BlockSpec((1, TOK_BLK, E), lambda b, i: (b, i, 0)),
        out_shape=jax.ShapeDtypeStruct((B, S, E), jnp.float32),
        compiler_params=pltpu.CompilerParams(
            dimension_semantics=("arbitrary", "arbitrary"),
        ),
    )(hidden_states, W)


# speedup vs baseline: 1.1956x; 1.1757x over previous
import jax
import jax.numpy as jnp
from jax.experimental import pallas as pl
from jax.experimental.pallas import tpu as pltpu

TOK_BLK = 2048


def _router_kernel(x_ref, w_ref, o_ref):
    o_ref[0] = jax.lax.dot_general(
        w_ref[...], x_ref[0],
        dimension_numbers=(((1,), (1,)), ((), ())),
        preferred_element_type=jnp.float32)


def kernel(hidden_states, W):
    B, S, H = hidden_states.shape
    E = W.shape[0]
    out_t = pl.pallas_call(
        _router_kernel,
        grid=(B, S // TOK_BLK),
        in_specs=[
            pl.BlockSpec((1, TOK_BLK, H), lambda b, i: (b, i, 0)),
            pl.BlockSpec((E, H), lambda b, i: (0, 0)),
        ],
        out_specs=pl.BlockSpec((1, E, TOK_BLK), lambda b, i: (b, 0, i)),
        out_shape=jax.ShapeDtypeStruct((B, E, S), jnp.float32),
        compiler_params=pltpu.CompilerParams(
            dimension_semantics=("arbitrary", "arbitrary"),
        ),
    )(hidden_states, W)
    return jnp.transpose(out_t, (0, 2, 1))


# speedup vs baseline: 1.2152x; 1.0164x over previous
import jax
import jax.numpy as jnp
from jax.experimental import pallas as pl
from jax.experimental.pallas import tpu as pltpu

TOK_BLK = 1024


def _router_kernel(x_ref, w_ref, o_ref):
    o_ref[0] = jax.lax.dot_general(
        w_ref[...], x_ref[0],
        dimension_numbers=(((1,), (1,)), ((), ())),
        preferred_element_type=jnp.float32)


def kernel(hidden_states, W):
    B, S, H = hidden_states.shape
    E = W.shape[0]
    out_t = pl.pallas_call(
        _router_kernel,
        grid=(B, S // TOK_BLK),
        in_specs=[
            pl.BlockSpec((1, TOK_BLK, H), lambda b, i: (b, i, 0)),
            pl.BlockSpec((E, H), lambda b, i: (0, 0)),
        ],
        out_specs=pl.BlockSpec((1, E, TOK_BLK), lambda b, i: (b, 0, i)),
        out_shape=jax.ShapeDtypeStruct((B, E, S), jnp.float32),
        compiler_params=pltpu.CompilerParams(
            dimension_semantics=("arbitrary", "arbitrary"),
        ),
    )(hidden_states, W)
    return jnp.transpose(out_t, (0, 2, 1))
